# baseline (speedup 1.0000x reference)
import functools

import jax
import jax.numpy as jnp
from jax import lax
from jax.experimental import pallas as pl
from jax.experimental.pallas import tpu as pltpu
from jax.experimental.pallas import tpu_sc as plsc

VOCAB = 1000
BT = 1024 * 50
NC, NS, L = 2, 16, 16
NW = NC * NS
PER_W = BT // NW
CHUNK = 64
NCHUNK = PER_W // CHUNK


def _lse_body(x_ref, o_ref):
    x = x_ref[...]
    m = jnp.max(x, axis=1)
    s = jnp.sum(jnp.exp(x - m[:, None]), axis=1)
    o_ref[...] = m + jnp.log(s)


def _row_lse(table_padded):
    return pl.pallas_call(
        _lse_body,
        out_shape=jax.ShapeDtypeStruct((VOCAB,), jnp.float32),
    )(table_padded)


def _loss_body(p_ref, o_ref):
    o_ref[0, 0] = jnp.sum(p_ref[...]) * (1.0 / BT)


def _loss_mean(parts):
    return pl.pallas_call(
        _loss_body,
        out_shape=jax.ShapeDtypeStruct((1, 1), jnp.float32),
        out_specs=pl.BlockSpec(memory_space=pltpu.MemorySpace.SMEM),
    )(parts)


_sc_mesh = plsc.VectorSubcoreMesh(core_axis_name="c", subcore_axis_name="s")


@functools.partial(
    pl.kernel,
    mesh=_sc_mesh,
    compiler_params=pltpu.CompilerParams(needs_layout_passes=False),
    out_type=[
        jax.ShapeDtypeStruct((BT, VOCAB), jnp.float32),
        jax.ShapeDtypeStruct((NW, L), jnp.float32),
    ],
    scratch_types=[
        pltpu.VMEM((CHUNK,), jnp.int32),
        pltpu.VMEM((CHUNK,), jnp.int32),
        pltpu.VMEM((CHUNK,), jnp.int32),
        pltpu.VMEM((CHUNK,), jnp.float32),
        pltpu.VMEM((VOCAB,), jnp.float32),
        pltpu.VMEM((L,), jnp.float32),
        pltpu.SemaphoreType.DMA,
        pltpu.SemaphoreType.DMA,
    ],
)
def _sc_gather(idx_hbm, tgt_hbm, table_hbm, tflat_hbm, lse_hbm,
               out_hbm, part_hbm,
               idx_v, tgt_v, flat_v, pick_v, lse_v, acc_v, sem, rsem):
    wid = lax.axis_index("s") * NC + lax.axis_index("c")
    base = wid * PER_W
    pltpu.sync_copy(lse_hbm, lse_v)

    def chunk_body(j, acc):
        off = base + j * CHUNK
        pltpu.sync_copy(idx_hbm.at[pl.ds(off, CHUNK)], idx_v)
        pltpu.sync_copy(tgt_hbm.at[pl.ds(off, CHUNK)], tgt_v)
        handles = []
        for k in range(CHUNK // L):
            iv = idx_v[pl.ds(k * L, L)]
            for r in range(L):
                s = iv[r]
                handles.append(pltpu.async_copy(
                    table_hbm.at[pl.ds(s, 1)],
                    out_hbm.at[pl.ds(off + k * L + r, 1)],
                    rsem))
        for k in range(CHUNK // L):
            sl = pl.ds(k * L, L)
            flat_v[sl] = idx_v[sl] * VOCAB + tgt_v[sl]
        pltpu.async_copy(tflat_hbm.at[flat_v], pick_v, sem).wait()
        for k in range(CHUNK // L):
            sl = pl.ds(k * L, L)
            lg = plsc.load_gather(lse_v, [idx_v[sl]])
            acc = acc + (lg - pick_v[sl])
        for h in handles:
            h.wait()
        return acc

    acc = lax.fori_loop(0, NCHUNK, chunk_body, jnp.zeros((L,), jnp.float32))
    acc_v[...] = acc
    pltpu.sync_copy(acc_v, part_hbm.at[wid])


def kernel(idx, targets, table):
    idxf = idx.reshape(-1).astype(jnp.int32)
    tgtf = targets.reshape(-1).astype(jnp.int32)
    table_padded = jnp.pad(table, ((0, 0), (0, 24)),
                           constant_values=-jnp.inf)
    lse = _row_lse(table_padded)
    logits, parts = _sc_gather(idxf, tgtf, table, table.reshape(-1), lse)
    loss = _loss_mean(parts)[0, 0]
    return logits, loss


# speedup vs baseline: 15.3402x; 15.3402x over previous
import functools

import jax
import jax.numpy as jnp
from jax import lax
from jax.experimental import pallas as pl
from jax.experimental.pallas import tpu as pltpu
from jax.experimental.pallas import tpu_sc as plsc

VOCAB = 1000
VPAD = 1024
ALIGNED = 896
TAIL = VOCAB - ALIGNED
BT = 1024 * 50
NC, NS, L = 2, 16, 16
NW = NC * NS
PER_W = BT // NW
CHUNK = 64
NCHUNK = PER_W // CHUNK


def _lse_body(x_ref, o_ref):
    x = x_ref[...]
    m = jnp.max(x, axis=1)
    s = jnp.sum(jnp.exp(x - m[:, None]), axis=1)
    o_ref[...] = m + jnp.log(s)


def _row_lse(table_padded):
    return pl.pallas_call(
        _lse_body,
        out_shape=jax.ShapeDtypeStruct((VOCAB,), jnp.float32),
    )(table_padded)


def _loss_body(p_ref, o_ref):
    o_ref[0, 0] = jnp.sum(p_ref[...]) * (1.0 / BT)


def _loss_mean(parts):
    return pl.pallas_call(
        _loss_body,
        out_shape=jax.ShapeDtypeStruct((1, 1), jnp.float32),
        out_specs=pl.BlockSpec(memory_space=pltpu.MemorySpace.SMEM),
    )(parts)


_sc_mesh = plsc.VectorSubcoreMesh(core_axis_name="c", subcore_axis_name="s")


@functools.partial(
    pl.kernel,
    mesh=_sc_mesh,
    compiler_params=pltpu.CompilerParams(needs_layout_passes=False),
    out_type=[
        jax.ShapeDtypeStruct((BT, VOCAB), jnp.float32),
        jax.ShapeDtypeStruct((NW, L), jnp.float32),
    ],
    scratch_types=[
        pltpu.VMEM((CHUNK,), jnp.int32),
        pltpu.VMEM((CHUNK,), jnp.int32),
        pltpu.VMEM((CHUNK, VPAD), jnp.float32),
        pltpu.VMEM((CHUNK, TAIL), jnp.float32),
        pltpu.VMEM((VOCAB,), jnp.float32),
        pltpu.VMEM((L,), jnp.float32),
        pltpu.SemaphoreType.DMA,
    ],
)
def _sc_gather(idx_hbm, tgt_hbm, table_hbm, lse_hbm,
               out_hbm, part_hbm,
               idx_v, tgt_v, rows_v, tail_v, lse_v, acc_v, sem):
    wid = lax.axis_index("s") * NC + lax.axis_index("c")
    base = wid * PER_W
    pltpu.sync_copy(lse_hbm, lse_v)
    lanes = lax.iota(jnp.int32, L)

    def chunk_body(j, acc):
        off = base + j * CHUNK
        pltpu.sync_copy(idx_hbm.at[pl.ds(off, CHUNK)], idx_v)
        pltpu.sync_copy(tgt_hbm.at[pl.ds(off, CHUNK)], tgt_v)
        pltpu.async_copy(table_hbm.at[idx_v], rows_v, sem).wait()
        pltpu.sync_copy(rows_v.at[:, pl.ds(0, ALIGNED)],
                        out_hbm.at[pl.ds(off, CHUNK), pl.ds(0, ALIGNED)])

        def row_body(r, _):
            for c in (0, 16, 32, 48, 64, 80, TAIL - L):
                tail_v[r, pl.ds(c, L)] = rows_v[r, pl.ds(ALIGNED + c, L)]
            return 0

        lax.fori_loop(0, CHUNK, row_body, 0)
        pltpu.sync_copy(tail_v,
                        out_hbm.at[pl.ds(off, CHUNK), pl.ds(ALIGNED, TAIL)])

        for k in range(CHUNK // L):
            sl = pl.ds(k * L, L)
            pick = plsc.load_gather(rows_v, [lanes + (k * L), tgt_v[sl]])
            lg = plsc.load_gather(lse_v, [idx_v[sl]])
            acc = acc + (lg - pick)
        return acc

    acc = lax.fori_loop(0, NCHUNK, chunk_body, jnp.zeros((L,), jnp.float32))
    acc_v[...] = acc
    pltpu.sync_copy(acc_v, part_hbm.at[wid])


def kernel(idx, targets, table):
    idxf = idx.reshape(-1).astype(jnp.int32)
    tgtf = targets.reshape(-1).astype(jnp.int32)
    table_padded = jnp.pad(table, ((0, 0), (0, VPAD - VOCAB)),
                           constant_values=-jnp.inf)
    lse = _row_lse(table_padded)
    logits, parts = _sc_gather(idxf, tgtf, table_padded, lse)
    loss = _loss_mean(parts)[0, 0]
    return logits, loss
